# baseline (speedup 1.0000x reference)
import functools

import jax
import jax.numpy as jnp
import numpy as np
from jax import lax
from jax.experimental import pallas as pl
from jax.experimental.pallas import tpu as pltpu
from jax.experimental.pallas import tpu_sc as plsc

_MASK = np.concatenate(
    [np.ones(25, dtype=bool), np.zeros(70, dtype=bool), np.ones(42, dtype=bool)]
)
_KEPT = np.nonzero(_MASK)[0].astype(np.int32)
_NKEPT = len(_KEPT)

_FRAMES, _PEOPLE, _KP, _DIMS = 8192, 4, 137, 3
_FT = _FRAMES // 128
_NW = 32

_QT = _FT // 4
_NUNITS = _NKEPT * 4
_MAXJ = -(-_NUNITS // _NW)


_NBUF = 3


def _body(in_hbm, out_hbm, b0, b1, b2, ls0, ls1, ls2, ss0, ss1, ss2):
    wid = lax.axis_index("s") * 2 + lax.axis_index("c")
    bufs, lsems, ssems = (b0, b1, b2), (ls0, ls1, ls2), (ss0, ss1, ss2)

    units = []
    for j in range(_MAXJ):
        u = wid + _NW * j
        k_out = u // 4
        q = u % 4
        kp = jnp.where(k_out >= 25, k_out + 70, k_out)
        b = j % _NBUF
        lds = [
            pltpu.make_async_copy(
                in_hbm.at[3 * kp + d, pl.ds(q * _QT, _QT)],
                bufs[b].at[:, d],
                lsems[b],
            )
            for d in (0, 1)
        ]
        sts = [
            pltpu.make_async_copy(
                bufs[b].at[:, :, p, :],
                out_hbm.at[p, k_out, pl.ds(q * _QT, _QT)],
                ssems[b],
            )
            for p in range(_PEOPLE)
        ]
        units.append((u, lds, sts))

    def guarded(j, fn):
        @pl.when(units[j][0] < _NUNITS)
        def _():
            fn()

    def start_loads(j):
        guarded(j, lambda: [c.start() for c in units[j][1]])

    start_loads(0)
    if _MAXJ > 1:
        start_loads(1)
    for j in range(_MAXJ):
        if j + 2 < _MAXJ:
            if j - 1 >= 0:
                guarded(j - 1, lambda j=j: [c.wait() for c in units[j - 1][2]])
            start_loads(j + 2)
        guarded(j, lambda j=j: ([c.wait() for c in units[j][1]],
                                [c.start() for c in units[j][2]]))

    for j in (_MAXJ - 3, _MAXJ - 2, _MAXJ - 1):
        if j >= 0:
            guarded(j, lambda j=j: [c.wait() for c in units[j][2]])


_sc_filter = functools.partial(
    pl.kernel,
    mesh=plsc.VectorSubcoreMesh(core_axis_name="c", subcore_axis_name="s"),
    out_type=jax.ShapeDtypeStruct((_PEOPLE, _NKEPT, _FT, 2, 128), jnp.float32),
    scratch_types=[
        pltpu.VMEM((_QT, 2, _PEOPLE, 128), jnp.float32),
        pltpu.VMEM((_QT, 2, _PEOPLE, 128), jnp.float32),
        pltpu.VMEM((_QT, 2, _PEOPLE, 128), jnp.float32),
        pltpu.SemaphoreType.DMA,
        pltpu.SemaphoreType.DMA,
        pltpu.SemaphoreType.DMA,
        pltpu.SemaphoreType.DMA,
        pltpu.SemaphoreType.DMA,
        pltpu.SemaphoreType.DMA,
    ],
)(_body)


def kernel(pose):
    phys_in = jnp.transpose(
        pose.reshape(_FT, 128, _PEOPLE, _KP, _DIMS), (3, 4, 0, 2, 1)
    ).reshape(_KP * _DIMS, _FT, _PEOPLE, 128)
    out_phys = _sc_filter(phys_in)
    return jnp.transpose(out_phys, (2, 4, 0, 1, 3)).reshape(
        _FRAMES, _PEOPLE, _NKEPT, 2
    )


# speedup vs baseline: 1.0679x; 1.0679x over previous
import functools

import jax
import jax.numpy as jnp
import numpy as np
from jax import lax
from jax.experimental import pallas as pl
from jax.experimental.pallas import tpu as pltpu
from jax.experimental.pallas import tpu_sc as plsc

_MASK = np.concatenate(
    [np.ones(25, dtype=bool), np.zeros(70, dtype=bool), np.ones(42, dtype=bool)]
)
_KEPT = np.nonzero(_MASK)[0].astype(np.int32)
_NKEPT = len(_KEPT)

_FRAMES, _PEOPLE, _KP, _DIMS = 8192, 4, 137, 3
_FT = _FRAMES // 128
_NW = 32

_NQ = 2
_QT = _FT // _NQ
_NUNITS = _NKEPT * _NQ
_MAXJ = -(-_NUNITS // _NW)


_NBUF = 3


def _body(in_hbm, out_hbm, b0, b1, b2, ls0, ls1, ls2, ss0, ss1, ss2):
    wid = lax.axis_index("s") * 2 + lax.axis_index("c")
    bufs, lsems, ssems = (b0, b1, b2), (ls0, ls1, ls2), (ss0, ss1, ss2)

    units = []
    for j in range(_MAXJ):
        u = wid + _NW * j
        k_out = u // _NQ
        q = u % _NQ
        kp = jnp.where(k_out >= 25, k_out + 70, k_out)
        b = j % _NBUF
        lds = [
            pltpu.make_async_copy(
                in_hbm.at[3 * kp + d, pl.ds(q * _QT, _QT)],
                bufs[b].at[:, d],
                lsems[b],
            )
            for d in (0, 1)
        ]
        sts = [
            pltpu.make_async_copy(
                bufs[b].at[:, :, p, :],
                out_hbm.at[p, k_out, pl.ds(q * _QT, _QT)],
                ssems[b],
            )
            for p in range(_PEOPLE)
        ]
        units.append((u, lds, sts))

    def guarded(j, fn):
        @pl.when(units[j][0] < _NUNITS)
        def _():
            fn()

    def start_loads(j):
        guarded(j, lambda: [c.start() for c in units[j][1]])

    start_loads(0)
    if _MAXJ > 1:
        start_loads(1)
    for j in range(_MAXJ):
        if j + 2 < _MAXJ:
            if j - 1 >= 0:
                guarded(j - 1, lambda j=j: [c.wait() for c in units[j - 1][2]])
            start_loads(j + 2)
        guarded(j, lambda j=j: ([c.wait() for c in units[j][1]],
                                [c.start() for c in units[j][2]]))

    for j in (_MAXJ - 3, _MAXJ - 2, _MAXJ - 1):
        if j >= 0:
            guarded(j, lambda j=j: [c.wait() for c in units[j][2]])


_sc_filter = functools.partial(
    pl.kernel,
    mesh=plsc.VectorSubcoreMesh(core_axis_name="c", subcore_axis_name="s"),
    out_type=jax.ShapeDtypeStruct((_PEOPLE, _NKEPT, _FT, 2, 128), jnp.float32),
    scratch_types=[
        pltpu.VMEM((_QT, 2, _PEOPLE, 128), jnp.float32),
        pltpu.VMEM((_QT, 2, _PEOPLE, 128), jnp.float32),
        pltpu.VMEM((_QT, 2, _PEOPLE, 128), jnp.float32),
        pltpu.SemaphoreType.DMA,
        pltpu.SemaphoreType.DMA,
        pltpu.SemaphoreType.DMA,
        pltpu.SemaphoreType.DMA,
        pltpu.SemaphoreType.DMA,
        pltpu.SemaphoreType.DMA,
    ],
)(_body)


def kernel(pose):
    phys_in = jnp.transpose(
        pose.reshape(_FT, 128, _PEOPLE, _KP, _DIMS), (3, 4, 0, 2, 1)
    ).reshape(_KP * _DIMS, _FT, _PEOPLE, 128)
    out_phys = _sc_filter(phys_in)
    return jnp.transpose(out_phys, (2, 4, 0, 1, 3)).reshape(
        _FRAMES, _PEOPLE, _NKEPT, 2
    )


# speedup vs baseline: 1.0731x; 1.0049x over previous
import functools

import jax
import jax.numpy as jnp
import numpy as np
from jax import lax
from jax.experimental import pallas as pl
from jax.experimental.pallas import tpu as pltpu
from jax.experimental.pallas import tpu_sc as plsc

_MASK = np.concatenate(
    [np.ones(25, dtype=bool), np.zeros(70, dtype=bool), np.ones(42, dtype=bool)]
)
_KEPT = np.nonzero(_MASK)[0].astype(np.int32)
_NKEPT = len(_KEPT)

_FRAMES, _PEOPLE, _KP, _DIMS = 8192, 4, 137, 3
_FT = _FRAMES // 128
_NW = 32

_NQ = 2
_QT = _FT // _NQ
_NUNITS = _NKEPT * _NQ
_MAXJ = -(-_NUNITS // _NW)


_NBUF = 3


def _body(in_hbm, out_hbm, b0, b1, b2, ls0, ls1, ls2, ss0, ss1, ss2):
    wid = lax.axis_index("s") * 2 + lax.axis_index("c")
    bufs, lsems, ssems = (b0, b1, b2), (ls0, ls1, ls2), (ss0, ss1, ss2)

    units = []
    for j in range(_MAXJ):
        u = wid + _NW * j
        k_out = u // _NQ
        q = u % _NQ
        kp = jnp.where(k_out >= 25, k_out + 70, k_out)
        b = j % _NBUF
        lds = [
            pltpu.make_async_copy(
                in_hbm.at[3 * kp + d, pl.ds(q * _QT, _QT)],
                bufs[b].at[:, d],
                lsems[b],
            )
            for d in (0, 1)
        ]
        sts = [
            pltpu.make_async_copy(
                bufs[b].at[:, :, p, :],
                out_hbm.at[p, k_out, pl.ds(q * _QT, _QT)],
                ssems[b],
            )
            for p in range(_PEOPLE)
        ]
        units.append((u, lds, sts))

    def guarded(j, fn):
        @pl.when(units[j][0] < _NUNITS)
        def _():
            fn()

    def start_loads(j):
        guarded(j, lambda: [c.start() for c in units[j][1]])

    start_loads(0)
    if _MAXJ > 1:
        start_loads(1)
    for j in range(_MAXJ):
        if j + 2 < _MAXJ:
            if j - 1 >= 0:
                guarded(j - 1, lambda j=j: [c.wait() for c in units[j - 1][2]])
            start_loads(j + 2)
        guarded(j, lambda j=j: ([c.wait() for c in units[j][1]],
                                [c.start() for c in units[j][2]]))

    for j in (_MAXJ - 3, _MAXJ - 2, _MAXJ - 1):
        if j >= 0:
            guarded(j, lambda j=j: [c.wait() for c in units[j][2]])


_sc_filter = functools.partial(
    pl.kernel,
    mesh=plsc.VectorSubcoreMesh(core_axis_name="c", subcore_axis_name="s"),
    out_type=jax.ShapeDtypeStruct((_PEOPLE, _NKEPT, _FT, 2, 128), jnp.float32),
    scratch_types=[
        pltpu.VMEM((_QT, 2, _PEOPLE, 128), jnp.float32),
        pltpu.VMEM((_QT, 2, _PEOPLE, 128), jnp.float32),
        pltpu.VMEM((_QT, 2, _PEOPLE, 128), jnp.float32),
        pltpu.SemaphoreType.DMA,
        pltpu.SemaphoreType.DMA,
        pltpu.SemaphoreType.DMA,
        pltpu.SemaphoreType.DMA,
        pltpu.SemaphoreType.DMA,
        pltpu.SemaphoreType.DMA,
    ],
    compiler_params=pltpu.CompilerParams(skip_device_barrier=True),
)(_body)


def kernel(pose):
    phys_in = jnp.transpose(
        pose.reshape(_FT, 128, _PEOPLE, _KP, _DIMS), (3, 4, 0, 2, 1)
    ).reshape(_KP * _DIMS, _FT, _PEOPLE, 128)
    out_phys = _sc_filter(phys_in)
    return jnp.transpose(out_phys, (2, 4, 0, 1, 3)).reshape(
        _FRAMES, _PEOPLE, _NKEPT, 2
    )
